# baseline (speedup 1.0000x reference)
import jax
import jax.numpy as jnp
from jax.experimental import pallas as pl
from jax.experimental.pallas import tpu as pltpu

B, L, D = 8, 2048, 128
NBUF = 3
TIME_MEAN, TIME_STD = 43.8756927994, 51.4811932987
DIST_MEAN, DIST_STD = 0.274716042312, 0.127051674693


def _shift_down(a):
    r = pltpu.roll(a, shift=1, axis=0)
    row = jax.lax.broadcasted_iota(jnp.int32, a.shape, 0)
    return jnp.where(row == 0, 0.0, r)


def _shift_up(a):
    return pltpu.roll(a, shift=a.shape[0] - 1, axis=0)


def _body(dis_ref, x_ref, wp1_ref, bp1_ref, ws1_ref, wn1_ref, b1_ref,
          wp3_ref, bp3_ref, ws3_ref, wn3_ref, b3_ref,
          out_ref, scratch_ref, sem):
    i = pl.program_id(0)
    slot = jax.lax.rem(i, NBUF)
    f32 = jnp.float32

    @pl.when(i >= NBUF)
    def _():
        pltpu.make_async_copy(scratch_ref.at[slot],
                              out_ref.at[i - NBUF],
                              sem.at[slot]).wait()

    x = x_ref[0]
    p1 = jax.nn.relu(jnp.dot(x, wp1_ref[...], preferred_element_type=f32)
                     + bp1_ref[...])
    q1 = jnp.dot(p1, wn1_ref[...], preferred_element_type=f32)
    h = (jnp.dot(x, ws1_ref[...], preferred_element_type=f32)
         + _shift_down(q1) + b1_ref[...])
    e1 = h * _shift_up(h)

    p3 = jax.nn.relu(jnp.dot(e1, wp3_ref[...], preferred_element_type=f32)
                     + bp3_ref[...])
    q3 = jnp.dot(p3, wn3_ref[...], preferred_element_type=f32)
    h2 = (jnp.dot(e1, ws3_ref[...], preferred_element_type=f32)
          + _shift_down(q3) + b3_ref[...])
    e2 = h2 * _shift_up(h2)
    scratch_ref[slot, :, :D] = e2[:L - 2, :]

    d = (dis_ref[0] - DIST_MEAN) / DIST_STD
    loc_row = (pltpu.roll(d, shift=L - 2, axis=1) - d - DIST_MEAN) / DIST_STD
    scratch_ref[slot, :, D:] = jnp.transpose(loc_row)[:L - 2, :]

    pltpu.make_async_copy(scratch_ref.at[slot], out_ref.at[i],
                          sem.at[slot]).start()

    @pl.when(i == B - 1)
    def _():
        for k in range(NBUF - 1):
            j = B - 1 - NBUF + 1 + k
            pltpu.make_async_copy(scratch_ref.at[jax.lax.rem(j, NBUF)],
                                  out_ref.at[j],
                                  sem.at[jax.lax.rem(j, NBUF)]).wait()
        pltpu.make_async_copy(scratch_ref.at[slot], out_ref.at[i],
                              sem.at[slot]).wait()


def kernel(timeid, current_tim, current_dis, loc, attr_t,
           W_pool1, b_pool1, W_self1, W_neigh1, b1,
           W_pool3, b_pool3, W_self3, W_neigh3, b3):
    dis_row = current_dis.reshape(B, 1, L)
    w_spec = pl.BlockSpec((D, D), lambda b: (0, 0))
    bias_spec = pl.BlockSpec((1, D), lambda b: (0, 0))

    return pl.pallas_call(
        _body,
        grid=(B,),
        in_specs=[
            pl.BlockSpec((1, 1, L), lambda b: (b, 0, 0)),
            pl.BlockSpec((1, L, D), lambda b: (b, 0, 0)),
            w_spec, bias_spec, w_spec, w_spec, bias_spec,
            w_spec, bias_spec, w_spec, w_spec, bias_spec,
        ],
        out_specs=pl.BlockSpec(memory_space=pl.ANY),
        out_shape=jax.ShapeDtypeStruct((B, L - 2, D + 1), jnp.float32),
        scratch_shapes=[
            pltpu.MemorySpace.VMEM((NBUF, L - 2, D + 1), jnp.float32),
            pltpu.SemaphoreType.DMA((NBUF,)),
        ],
    )(dis_row, loc,
      W_pool1, b_pool1.reshape(1, D), W_self1, W_neigh1, b1.reshape(1, D),
      W_pool3, b_pool3.reshape(1, D), W_self3, W_neigh3, b3.reshape(1, D))


# speedup vs baseline: 1.0553x; 1.0553x over previous
import jax
import jax.numpy as jnp
from jax.experimental import pallas as pl
from jax.experimental.pallas import tpu as pltpu

B, L, D = 8, 2048, 128
G = 2
NP = B // G
NBUF = 3
TIME_MEAN, TIME_STD = 43.8756927994, 51.4811932987
DIST_MEAN, DIST_STD = 0.274716042312, 0.127051674693


def _row_iota():
    return jax.lax.broadcasted_iota(jnp.int32, (G * L, 1), 0)


def _shift_down(a):
    r = pltpu.roll(a, shift=1, axis=0)
    return jnp.where(jax.lax.rem(_row_iota(), L) == 0, 0.0, r)


def _shift_up(a):
    return pltpu.roll(a, shift=a.shape[0] - 1, axis=0)


def _body(dis_ref, x_ref, wp1_ref, bp1_ref, ws1_ref, wn1_ref, b1_ref,
          wp3_ref, bp3_ref, ws3_ref, wn3_ref, b3_ref,
          out_ref, scratch_ref, sem):
    i = pl.program_id(0)
    slot = jax.lax.rem(i, NBUF)
    f32 = jnp.float32

    @pl.when(i >= NBUF)
    def _():
        pltpu.make_async_copy(scratch_ref.at[slot],
                              out_ref.at[pl.ds((i - NBUF) * G, G)],
                              sem.at[slot]).wait()

    x = x_ref[...].reshape(G * L, D)
    p1 = jax.nn.relu(jnp.dot(x, wp1_ref[...], preferred_element_type=f32)
                     + bp1_ref[...])
    q1 = jnp.dot(p1, wn1_ref[...], preferred_element_type=f32)
    h = (jnp.dot(x, ws1_ref[...], preferred_element_type=f32)
         + _shift_down(q1) + b1_ref[...])
    e1 = h * _shift_up(h)

    p3 = jax.nn.relu(jnp.dot(e1, wp3_ref[...], preferred_element_type=f32)
                     + bp3_ref[...])
    q3 = jnp.dot(p3, wn3_ref[...], preferred_element_type=f32)
    h2 = (jnp.dot(e1, ws3_ref[...], preferred_element_type=f32)
          + _shift_down(q3) + b3_ref[...])
    e2 = h2 * _shift_up(h2)

    d = (dis_ref[...].reshape(G, L) - DIST_MEAN) / DIST_STD
    loc_row = (pltpu.roll(d, shift=L - 2, axis=1) - d - DIST_MEAN) / DIST_STD
    for g in range(G):
        scratch_ref[slot, g, :, :D] = e2[g * L:g * L + L - 2, :]
        scratch_ref[slot, g, :, D:] = (
            jnp.transpose(loc_row[g:g + 1, :])[:L - 2, :])

    pltpu.make_async_copy(scratch_ref.at[slot],
                          out_ref.at[pl.ds(i * G, G)],
                          sem.at[slot]).start()

    @pl.when(i == NP - 1)
    def _():
        for j in range(max(0, NP - NBUF), NP):
            pltpu.make_async_copy(scratch_ref.at[jax.lax.rem(jnp.int32(j),
                                                             NBUF)],
                                  out_ref.at[pl.ds(j * G, G)],
                                  sem.at[jax.lax.rem(jnp.int32(j),
                                                     NBUF)]).wait()


def kernel(timeid, current_tim, current_dis, loc, attr_t,
           W_pool1, b_pool1, W_self1, W_neigh1, b1,
           W_pool3, b_pool3, W_self3, W_neigh3, b3):
    dis_row = current_dis.reshape(B, 1, L)
    w_spec = pl.BlockSpec((D, D), lambda b: (0, 0))
    bias_spec = pl.BlockSpec((1, D), lambda b: (0, 0))

    return pl.pallas_call(
        _body,
        grid=(NP,),
        in_specs=[
            pl.BlockSpec((G, 1, L), lambda b: (b, 0, 0)),
            pl.BlockSpec((G, L, D), lambda b: (b, 0, 0)),
            w_spec, bias_spec, w_spec, w_spec, bias_spec,
            w_spec, bias_spec, w_spec, w_spec, bias_spec,
        ],
        out_specs=pl.BlockSpec(memory_space=pl.ANY),
        out_shape=jax.ShapeDtypeStruct((B, L - 2, D + 1), jnp.float32),
        scratch_shapes=[
            pltpu.MemorySpace.VMEM((NBUF, G, L - 2, D + 1), jnp.float32),
            pltpu.SemaphoreType.DMA((NBUF,)),
        ],
    )(dis_row, loc,
      W_pool1, b_pool1.reshape(1, D), W_self1, W_neigh1, b1.reshape(1, D),
      W_pool3, b_pool3.reshape(1, D), W_self3, W_neigh3, b3.reshape(1, D))
